# baseline (speedup 1.0000x reference)
import jax
import jax.numpy as jnp
from jax.experimental import pallas as pl
from jax.experimental.pallas import tpu as pltpu

_N_DMA = 32
_ROWS = 16384
_COLS = 2048


def _copy_body(x_ref, o_ref, sems):
    rows_per = _ROWS // _N_DMA
    copies = []
    for i in range(_N_DMA):
        sl = pl.ds(i * rows_per, rows_per)
        copies.append(pltpu.make_async_copy(x_ref.at[sl], o_ref.at[sl], sems.at[i]))
    for c in copies:
        c.start()
    for c in copies:
        c.wait()


def kernel(x):
    x2 = x.reshape(_ROWS, _COLS)
    out = pl.pallas_call(
        _copy_body,
        out_shape=jax.ShapeDtypeStruct((_ROWS, _COLS), x.dtype),
        in_specs=[pl.BlockSpec(memory_space=pl.ANY)],
        out_specs=pl.BlockSpec(memory_space=pl.ANY),
        scratch_shapes=[pltpu.SemaphoreType.DMA((_N_DMA,))],
    )(x2)
    return out.reshape(x.shape)


# speedup vs baseline: 48.0899x; 48.0899x over previous
import jax
import jax.numpy as jnp
from jax.experimental import pallas as pl
from jax.experimental.pallas import tpu as pltpu

_ROWS = 16384
_COLS = 2048
_BLOCK_ROWS = 512


def _copy_body(x_ref, o_ref):
    o_ref[...] = x_ref[...]


def kernel(x):
    x2 = x.reshape(_ROWS, _COLS)
    out = pl.pallas_call(
        _copy_body,
        grid=(_ROWS // _BLOCK_ROWS,),
        in_specs=[pl.BlockSpec((_BLOCK_ROWS, _COLS), lambda i: (i, 0))],
        out_specs=pl.BlockSpec((_BLOCK_ROWS, _COLS), lambda i: (i, 0)),
        out_shape=jax.ShapeDtypeStruct((_ROWS, _COLS), x.dtype),
        compiler_params=pltpu.CompilerParams(
            dimension_semantics=("arbitrary",),
        ),
    )(x2)
    return out.reshape(x.shape)


# speedup vs baseline: 49.0668x; 1.0203x over previous
import jax
import jax.numpy as jnp
from jax.experimental import pallas as pl
from jax.experimental.pallas import tpu as pltpu

_ROWS = 16384
_COLS = 2048
_BLOCK_ROWS = 1024


def _copy_body(x_ref, o_ref):
    o_ref[...] = x_ref[...]


def kernel(x):
    x2 = x.reshape(_ROWS, _COLS)
    out = pl.pallas_call(
        _copy_body,
        grid=(_ROWS // _BLOCK_ROWS,),
        in_specs=[pl.BlockSpec((_BLOCK_ROWS, _COLS), lambda i: (i, 0))],
        out_specs=pl.BlockSpec((_BLOCK_ROWS, _COLS), lambda i: (i, 0)),
        out_shape=jax.ShapeDtypeStruct((_ROWS, _COLS), x.dtype),
        compiler_params=pltpu.CompilerParams(
            dimension_semantics=("arbitrary",),
        ),
    )(x2)
    return out.reshape(x.shape)
